# baseline (speedup 1.0000x reference)
import functools

import jax
import jax.numpy as jnp
from jax import lax
from jax.experimental import pallas as pl
from jax.experimental.pallas import tpu as pltpu
from jax.experimental.pallas import tpu_sc as plsc

DIM = 64
NC = 2
NS = 16
L = 16
NW = NC * NS
CHUNK = 128


def _rsqrt(x):
    i = plsc.bitcast(x, jnp.int32)
    i = jnp.int32(0x5F3759DF) - lax.shift_right_logical(i, 1)
    y = plsc.bitcast(i, jnp.float32)
    for _ in range(3):
        y = y * (1.5 - 0.5 * x * y * y)
    return y


def _inv_norm(ss):
    n = ss * _rsqrt(ss)
    return 1.0 / jnp.maximum(n, 1e-12)


def _pair_row(v):
    return lax.shift_right_logical(v, 1)


def _pair_off(v):
    return lax.shift_left(v & 1, 6)


def _make_score_kernel(B):
    bpw = B // NW
    nchunks = bpw // CHUNK
    groups = CHUNK // L
    mesh = plsc.VectorSubcoreMesh(
        core_axis_name="c", subcore_axis_name="s", num_cores=NC,
        num_subcores=NS)

    @functools.partial(
        pl.kernel,
        out_type=jax.ShapeDtypeStruct((B,), jnp.float32),
        mesh=mesh,
        compiler_params=pltpu.CompilerParams(
            needs_layout_passes=False, use_tc_tiling_on_sc=True),
        scratch_types=[
            pltpu.VMEM((bpw,), jnp.int32),
            pltpu.VMEM((bpw,), jnp.int32),
            pltpu.VMEM((bpw,), jnp.int32),
            pltpu.VMEM((bpw,), jnp.int32),
            pltpu.VMEM((bpw,), jnp.int32),
            pltpu.VMEM((bpw,), jnp.int32),
            pltpu.VMEM((2 * CHUNK, 2 * DIM), jnp.float32),
            pltpu.VMEM((2 * CHUNK, 2 * DIM), jnp.float32),
            pltpu.VMEM((2 * CHUNK, 2 * DIM), jnp.float32),
            pltpu.VMEM((bpw,), jnp.float32),
            pltpu.SemaphoreType.DMA,
        ],
    )
    def k(h_hbm, r_hbm, t_hbm, pe_hbm, pr_hbm, out_hbm,
          hidx, tidx, ridx, hrow, trow, rrow,
          hbuf, tbuf, rbuf, outv, sem):
        wid = lax.axis_index("s") * NC + lax.axis_index("c")
        base = wid * bpw
        iota = lax.iota(jnp.int32, L)

        pltpu.sync_copy(h_hbm.at[pl.ds(base, bpw)], hidx)
        pltpu.sync_copy(t_hbm.at[pl.ds(base, bpw)], tidx)
        pltpu.sync_copy(r_hbm.at[pl.ds(base, bpw)], ridx)

        def row_body(g, carry):
            sl = pl.ds(g * L, L)
            hrow[sl] = _pair_row(hidx[sl])
            trow[sl] = _pair_row(tidx[sl])
            rrow[sl] = _pair_row(ridx[sl])
            return carry

        lax.fori_loop(0, bpw // L, row_body, 0)

        def fire(c, slot):
            sl = pl.ds(c * CHUNK, CHUNK)
            dsl = pl.ds(slot * CHUNK, CHUNK)
            return (
                pltpu.async_copy(pe_hbm.at[hrow.at[sl]], hbuf.at[dsl], sem),
                pltpu.async_copy(pe_hbm.at[trow.at[sl]], tbuf.at[dsl], sem),
                pltpu.async_copy(pr_hbm.at[rrow.at[sl]], rbuf.at[dsl], sem),
            )

        def compute(c, slot):
            def group_body(g, carry):
                jj = iota + (slot * CHUNK + g * L)
                gsl = pl.ds(c * CHUNK + g * L, L)
                ho = _pair_off(hidx[gsl])
                to = _pair_off(tidx[gsl])
                ro = _pair_off(ridx[gsl])
                zero = jnp.zeros((L,), jnp.float32)

                def d_body(o, acc):
                    ssh, sst, ssr, dht, dhr, dtr = acc
                    for dd in range(8):
                        d = o * 8 + dd
                        hv = plsc.load_gather(hbuf, [jj, ho + d])
                        tv = plsc.load_gather(tbuf, [jj, to + d])
                        rv = plsc.load_gather(rbuf, [jj, ro + d])
                        ssh += hv * hv
                        sst += tv * tv
                        ssr += rv * rv
                        dht += hv * tv
                        dhr += hv * rv
                        dtr += tv * rv
                    return ssh, sst, ssr, dht, dhr, dtr

                ssh, sst, ssr, dht, dhr, dtr = lax.fori_loop(
                    0, DIM // 8, d_body, (zero,) * 6)
                a = _inv_norm(ssh)
                b = _inv_norm(sst)
                cc = _inv_norm(ssr)
                q = (ssh * a * a + sst * b * b + ssr * cc * cc
                     - 2.0 * ((a * b) * dht - (a * cc) * dhr + (b * cc) * dtr))
                qm = jnp.maximum(q, 0.0)
                outv[pl.ds(c * CHUNK + g * L, L)] = qm * _rsqrt(qm)
                return carry

            lax.fori_loop(0, groups, group_body, 0)

        pend = fire(0, 0)
        for c in range(nchunks):
            slot = c % 2
            nxt = None
            if c + 1 < nchunks:
                nxt = fire(c + 1, (c + 1) % 2)
            for cp in pend:
                cp.wait()
            compute(c, slot)
            pend = nxt

        pltpu.sync_copy(outv, out_hbm.at[pl.ds(base, bpw)])

    return k


def kernel(h, r, t, emb_entity, emb_relation):
    h = h.astype(jnp.int32)
    r = r.astype(jnp.int32)
    t = t.astype(jnp.int32)
    pe = emb_entity.reshape(-1, 2 * DIM)
    pr = emb_relation.reshape(-1, 2 * DIM)
    score = _make_score_kernel(h.shape[0])
    return score(h, r, t, pe, pr)


# speedup vs baseline: 2.3812x; 2.3812x over previous
import functools

import jax
import jax.numpy as jnp
from jax import lax
from jax.experimental import pallas as pl
from jax.experimental.pallas import tpu as pltpu
from jax.experimental.pallas import tpu_sc as plsc

DIM = 64
NC = 2
NS = 16
L = 16
NW = NC * NS
CHUNK = 128


PBLK = 32768


def _pair_body(x_ref, o_ref):
    x = x_ref[...]
    o_ref[:, 0:DIM] = x[:, 0:PBLK // 2].T
    o_ref[:, DIM:2 * DIM] = x[:, PBLK // 2:PBLK].T


def _pairify_ent(xT):
    v = xT.shape[1]
    nblocks = (v + PBLK - 1) // PBLK
    return pl.pallas_call(
        _pair_body,
        grid=(nblocks,),
        in_specs=[pl.BlockSpec((DIM, PBLK), lambda i: (0, i))],
        out_specs=pl.BlockSpec((PBLK // 2, 2 * DIM), lambda i: (i, 0)),
        out_shape=jax.ShapeDtypeStruct(
            (nblocks * (PBLK // 2), 2 * DIM), jnp.float32),
    )(xT)


def _rel_pair_body(x_ref, o_ref):
    x = x_ref[...]
    half = x.shape[1] // 2
    o_ref[:, 0:DIM] = x[:, 0:half].T
    o_ref[:, DIM:2 * DIM] = x[:, half:2 * half].T


def _pairify_rel(xT):
    v = xT.shape[1]
    return pl.pallas_call(
        _rel_pair_body,
        grid=(1,),
        in_specs=[pl.BlockSpec((DIM, v), lambda i: (0, 0))],
        out_specs=pl.BlockSpec((v // 2, 2 * DIM), lambda i: (0, 0)),
        out_shape=jax.ShapeDtypeStruct((v // 2, 2 * DIM), jnp.float32),
    )(xT)


def _rsqrt(x):
    i = plsc.bitcast(x, jnp.int32)
    i = jnp.int32(0x5F3759DF) - lax.shift_right_logical(i, 1)
    y = plsc.bitcast(i, jnp.float32)
    for _ in range(3):
        y = y * (1.5 - 0.5 * x * y * y)
    return y


def _inv_norm(ss):
    n = ss * _rsqrt(ss)
    return 1.0 / jnp.maximum(n, 1e-12)


_PSH = PBLK.bit_length() - 1
_HMASK = PBLK // 2 - 1


def _ent_row(v):
    return lax.shift_left(lax.shift_right_logical(v, _PSH), _PSH - 1) + (v & _HMASK)


def _ent_off(v):
    return lax.shift_left(lax.shift_right_logical(v, _PSH - 1) & 1, 6)


def _make_score_kernel(B, rel_half):
    bpw = B // NW
    nchunks = bpw // CHUNK
    groups = CHUNK // L
    mesh = plsc.VectorSubcoreMesh(
        core_axis_name="c", subcore_axis_name="s", num_cores=NC,
        num_subcores=NS)

    @functools.partial(
        pl.kernel,
        out_type=jax.ShapeDtypeStruct((B,), jnp.float32),
        mesh=mesh,
        compiler_params=pltpu.CompilerParams(
            needs_layout_passes=False, use_tc_tiling_on_sc=True),
        scratch_types=[
            pltpu.VMEM((bpw,), jnp.int32),
            pltpu.VMEM((bpw,), jnp.int32),
            pltpu.VMEM((bpw,), jnp.int32),
            pltpu.VMEM((bpw,), jnp.int32),
            pltpu.VMEM((bpw,), jnp.int32),
            pltpu.VMEM((bpw,), jnp.int32),
            pltpu.VMEM((2 * CHUNK, 2 * DIM), jnp.float32),
            pltpu.VMEM((2 * CHUNK, 2 * DIM), jnp.float32),
            pltpu.VMEM((2 * CHUNK, 2 * DIM), jnp.float32),
            pltpu.VMEM((bpw,), jnp.float32),
            pltpu.SemaphoreType.DMA,
        ],
    )
    def k(h_hbm, r_hbm, t_hbm, pe_hbm, pr_hbm, out_hbm,
          hidx, tidx, ridx, hrow, trow, rrow,
          hbuf, tbuf, rbuf, outv, sem):
        wid = lax.axis_index("s") * NC + lax.axis_index("c")
        base = wid * bpw
        iota = lax.iota(jnp.int32, L)

        pltpu.sync_copy(h_hbm.at[pl.ds(base, bpw)], hidx)
        pltpu.sync_copy(t_hbm.at[pl.ds(base, bpw)], tidx)
        pltpu.sync_copy(r_hbm.at[pl.ds(base, bpw)], ridx)

        def row_body(g, carry):
            sl = pl.ds(g * L, L)
            hrow[sl] = _ent_row(hidx[sl])
            trow[sl] = _ent_row(tidx[sl])
            rv = ridx[sl]
            rrow[sl] = jnp.where(rv >= rel_half, rv - rel_half, rv)
            return carry

        lax.fori_loop(0, bpw // L, row_body, 0)

        def fire(c, slot):
            sl = pl.ds(c * CHUNK, CHUNK)
            dsl = pl.ds(slot * CHUNK, CHUNK)
            return (
                pltpu.async_copy(pe_hbm.at[hrow.at[sl]], hbuf.at[dsl], sem),
                pltpu.async_copy(pe_hbm.at[trow.at[sl]], tbuf.at[dsl], sem),
                pltpu.async_copy(pr_hbm.at[rrow.at[sl]], rbuf.at[dsl], sem),
            )

        def compute(c, slot):
            def group_body(g, carry):
                jj = iota + (slot * CHUNK + g * L)
                gsl = pl.ds(c * CHUNK + g * L, L)
                ho = _ent_off(hidx[gsl])
                to = _ent_off(tidx[gsl])
                ro = jnp.where(ridx[gsl] >= rel_half, DIM, 0)
                zero = jnp.zeros((L,), jnp.float32)

                def d_body(o, acc):
                    ssh, sst, ssr, dht, dhr, dtr = acc
                    for dd in range(8):
                        d = o * 8 + dd
                        hv = plsc.load_gather(hbuf, [jj, ho + d])
                        tv = plsc.load_gather(tbuf, [jj, to + d])
                        rv = plsc.load_gather(rbuf, [jj, ro + d])
                        ssh += hv * hv
                        sst += tv * tv
                        ssr += rv * rv
                        dht += hv * tv
                        dhr += hv * rv
                        dtr += tv * rv
                    return ssh, sst, ssr, dht, dhr, dtr

                ssh, sst, ssr, dht, dhr, dtr = lax.fori_loop(
                    0, DIM // 8, d_body, (zero,) * 6)
                a = _inv_norm(ssh)
                b = _inv_norm(sst)
                cc = _inv_norm(ssr)
                q = (ssh * a * a + sst * b * b + ssr * cc * cc
                     - 2.0 * ((a * b) * dht - (a * cc) * dhr + (b * cc) * dtr))
                qm = jnp.maximum(q, 0.0)
                outv[pl.ds(c * CHUNK + g * L, L)] = qm * _rsqrt(qm)
                return carry

            lax.fori_loop(0, groups, group_body, 0)

        pend = fire(0, 0)
        for c in range(nchunks):
            slot = c % 2
            nxt = None
            if c + 1 < nchunks:
                nxt = fire(c + 1, (c + 1) % 2)
            for cp in pend:
                cp.wait()
            compute(c, slot)
            pend = nxt

        pltpu.sync_copy(outv, out_hbm.at[pl.ds(base, bpw)])

    return k


def kernel(h, r, t, emb_entity, emb_relation):
    h = h.astype(jnp.int32)
    r = r.astype(jnp.int32)
    t = t.astype(jnp.int32)
    pe = _pairify_ent(emb_entity.T)
    pr = _pairify_rel(emb_relation.T)
    score = _make_score_kernel(h.shape[0], emb_relation.shape[0] // 2)
    return score(h, r, t, pe, pr)
